# baseline (speedup 1.0000x reference)
import functools

import jax
import jax.numpy as jnp
from jax import lax
from jax.experimental import pallas as pl
from jax.experimental.pallas import tpu as pltpu
from jax.experimental.pallas import tpu_sc as plsc

_N = 4096
_D_IN = 256
_D_OUT = 256
_BM = 512


def _fused_body(inputs_ref, weight_ref, adj_ref, bias_ref, out_ref, support_ref):
    @pl.when(pl.program_id(0) == 0)
    def _():
        support_ref[...] = jnp.dot(
            inputs_ref[...], weight_ref[...], preferred_element_type=jnp.float32
        )

    a = adj_ref[...].astype(jnp.bfloat16)
    s = support_ref[...].astype(jnp.bfloat16)
    acc = jnp.dot(a, s, preferred_element_type=jnp.float32)
    out_ref[...] = acc + bias_ref[...]


_NW = 32
_SC_ROWS = 32
_SC_CHUNK = 8


def _sc_probe(adj):
    mesh = plsc.VectorSubcoreMesh(core_axis_name="c", subcore_axis_name="s")

    @functools.partial(
        pl.kernel,
        mesh=mesh,
        out_type=jax.ShapeDtypeStruct((_NW, 16), jnp.float32),
        scratch_types=[
            pltpu.VMEM((_SC_CHUNK, _N), jnp.float32),
            pltpu.VMEM((16,), jnp.float32),
        ],
    )
    def sck(adj_hbm, out_hbm, buf, accv):
        wid = lax.axis_index("s") * 2 + lax.axis_index("c")
        base = wid * _SC_ROWS

        def body(k, acc):
            pltpu.sync_copy(adj_hbm.at[pl.ds(base + k * _SC_CHUNK, _SC_CHUNK)], buf)
            return acc + buf[0, 0:16]

        acc = lax.fori_loop(0, _SC_ROWS // _SC_CHUNK, body, jnp.zeros((16,), jnp.float32))
        accv[...] = acc
        pltpu.sync_copy(accv, out_hbm.at[wid])

    return sck(adj)


def kernel(inputs, adj, weight, bias):
    bias2d = bias.reshape(1, _D_OUT)
    sc_zero = jnp.sum(_sc_probe(adj)) * 0.0
    grid = (_N // _BM,)
    out = pl.pallas_call(
        _fused_body,
        grid=grid,
        in_specs=[
            pl.BlockSpec((_N, _D_IN), lambda i: (0, 0)),
            pl.BlockSpec((_D_IN, _D_OUT), lambda i: (0, 0)),
            pl.BlockSpec((_BM, _N), lambda i: (i, 0)),
            pl.BlockSpec((1, _D_OUT), lambda i: (0, 0)),
        ],
        out_specs=pl.BlockSpec((_BM, _D_OUT), lambda i: (i, 0)),
        out_shape=jax.ShapeDtypeStruct((_N, _D_OUT), jnp.float32),
        scratch_shapes=[pltpu.VMEM((_N, _D_OUT), jnp.float32)],
    )(inputs, weight, adj, bias2d)
    return out + sc_zero


# speedup vs baseline: 1.9433x; 1.9433x over previous
import functools

import jax
import jax.numpy as jnp
from jax.experimental import pallas as pl
from jax.experimental.pallas import tpu as pltpu

_N = 4096
_D_IN = 256
_D_OUT = 256
_BM = 512
_NH = _N // 2


def _fused_body(inputs_ref, weight_ref, adj_l_ref, adj_r_ref, bias_ref, out_ref, support_ref):
    @pl.when(pl.program_id(0) == 0)
    def _():
        support_ref[...] = jnp.dot(
            inputs_ref[...], weight_ref[...], preferred_element_type=jnp.float32
        )

    a_l = adj_l_ref[...].astype(jnp.bfloat16)
    a_r = adj_r_ref[...].astype(jnp.bfloat16)
    s = support_ref[...].astype(jnp.bfloat16)
    acc = jnp.dot(a_l, s[:_NH], preferred_element_type=jnp.float32)
    acc = acc + jnp.dot(a_r, s[_NH:], preferred_element_type=jnp.float32)
    out_ref[...] = acc + bias_ref[...]


def kernel(inputs, adj, weight, bias):
    bias2d = bias.reshape(1, _D_OUT)
    grid = (_N // _BM,)
    out = pl.pallas_call(
        _fused_body,
        grid=grid,
        in_specs=[
            pl.BlockSpec((_N, _D_IN), lambda i: (0, 0)),
            pl.BlockSpec((_D_IN, _D_OUT), lambda i: (0, 0)),
            pl.BlockSpec((_BM, _NH), lambda i: (i, 0)),
            pl.BlockSpec((_BM, _NH), lambda i: (i, 1)),
            pl.BlockSpec((1, _D_OUT), lambda i: (0, 0)),
        ],
        out_specs=pl.BlockSpec((_BM, _D_OUT), lambda i: (i, 0)),
        out_shape=jax.ShapeDtypeStruct((_N, _D_OUT), jnp.float32),
        scratch_shapes=[pltpu.VMEM((_N, _D_OUT), jnp.float32)],
    )(inputs, weight, adj, adj, bias2d)
    return out
